# baseline (speedup 1.0000x reference)
import jax
import jax.numpy as jnp
from jax import lax
from jax.experimental import pallas as pl
from jax.experimental.pallas import tpu as pltpu
from jax.experimental.pallas import tpu_sc as plsc

VOCAB = 1000000
EMBED_DIM = 64
HIDDEN_DIM = 128
OUT_DIM = 128
BATCH = 16384

_ROWS_PER_TILE = 16
_NMAJ = VOCAB // _ROWS_PER_TILE

_NC = 2
_NS = 16
_NW = _NC * _NS
_B_PER_W = BATCH // _NW
_CHUNK = 64
_NCHUNKS = _B_PER_W // _CHUNK


def _gather_body(tbl_hbm, idx_hbm, out_hbm, idx_v, m_v, buf, zbuf, sem):
    wid = lax.axis_index("s") * _NC + lax.axis_index("c")
    base = wid * _B_PER_W
    pltpu.sync_copy(idx_hbm.at[pl.ds(base, _B_PER_W)], idx_v)
    lane = lax.iota(jnp.int32, 16)

    def fill_m(j, carry):
        m_v[pl.ds(j * 16, 16)] = lax.shift_right_logical(idx_v[pl.ds(j * 16, 16)], 4)
        return carry

    lax.fori_loop(0, _B_PER_W // 16, fill_m, 0)

    def chunk(g, carry):
        pltpu.async_copy(tbl_hbm.at[m_v.at[pl.ds(g * _CHUNK, _CHUNK)]], buf, sem).wait()
        for j in range(_CHUNK // 16):
            iv = idx_v[pl.ds(g * _CHUNK + j * 16, 16)]
            s_vec = lax.shift_right_logical(iv, 1) & 7
            c_vec = iv & 1
            for l in range(16):
                pick = lane == l
                s = jnp.sum(jnp.where(pick, s_vec, 0))
                c = jnp.sum(jnp.where(pick, c_vec, 0))
                row = j * 16 + l
                for k in range(EMBED_DIM // 16):
                    zbuf[row, pl.ds(k * 16, 16)] = buf[row, s, pl.ds(c * 64 + k * 16, 16)]
        pltpu.sync_copy(zbuf, out_hbm.at[pl.ds(base + g * _CHUNK, _CHUNK)])
        return carry

    lax.fori_loop(0, _NCHUNKS, chunk, 0)


_sc_gather = pl.kernel(
    _gather_body,
    out_type=jax.ShapeDtypeStruct((BATCH, EMBED_DIM), jnp.float32),
    mesh=plsc.VectorSubcoreMesh(
        core_axis_name="c", subcore_axis_name="s", num_cores=_NC, num_subcores=_NS
    ),
    scratch_types=[
        pltpu.VMEM((_B_PER_W,), jnp.int32),
        pltpu.VMEM((_B_PER_W,), jnp.int32),
        pltpu.VMEM((_CHUNK, 8, 128), jnp.float32),
        pltpu.VMEM((_CHUNK, EMBED_DIM), jnp.float32),
        pltpu.SemaphoreType.DMA,
    ],
    compiler_params=pltpu.CompilerParams(needs_layout_passes=False),
)

_MLP_BLOCK = 2048
_GRID = BATCH // _MLP_BLOCK


def _mlp_body(z_ref, w1_ref, b1_ref, w2_ref, b2_ref, o_ref):
    h = jnp.dot(z_ref[...], w1_ref[...], preferred_element_type=jnp.float32)
    h = jnp.maximum(h + b1_ref[...], 0.0)
    o_ref[...] = (
        jnp.dot(h, w2_ref[...], preferred_element_type=jnp.float32) + b2_ref[...]
    )


def _tc_mlp(z, W1, b1, W2, b2):
    return pl.pallas_call(
        _mlp_body,
        grid=(_GRID,),
        in_specs=[
            pl.BlockSpec((_MLP_BLOCK, EMBED_DIM), lambda i: (i, 0)),
            pl.BlockSpec((EMBED_DIM, HIDDEN_DIM), lambda i: (0, 0)),
            pl.BlockSpec((1, HIDDEN_DIM), lambda i: (0, 0)),
            pl.BlockSpec((HIDDEN_DIM, OUT_DIM), lambda i: (0, 0)),
            pl.BlockSpec((1, OUT_DIM), lambda i: (0, 0)),
        ],
        out_specs=pl.BlockSpec((_MLP_BLOCK, OUT_DIM), lambda i: (i, 0)),
        out_shape=jax.ShapeDtypeStruct((BATCH, OUT_DIM), jnp.float32),
    )(z, W1, b1.reshape(1, HIDDEN_DIM), W2, b2.reshape(1, OUT_DIM))


def kernel(x, emb, W1, b1, W2, b2):
    idx = x.astype(jnp.int32)
    embc = emb.reshape(_NMAJ, 8, 128)
    z = _sc_gather(embc, idx)
    return _tc_mlp(z, W1, b1, W2, b2)


# speedup vs baseline: 2.3124x; 2.3124x over previous
import jax
import jax.numpy as jnp
from jax import lax
from jax.experimental import pallas as pl
from jax.experimental.pallas import tpu as pltpu
from jax.experimental.pallas import tpu_sc as plsc

VOCAB = 1000000
EMBED_DIM = 64
HIDDEN_DIM = 128
OUT_DIM = 128
BATCH = 16384

_TILE = 8
_NTILES = VOCAB // _TILE

_NC = 2
_NS = 16
_NW = _NC * _NS
_B_PER_W = BATCH // _NW
_CHUNK = 16
_NPAIR = _B_PER_W // (2 * _CHUNK)


def _gather_body(tbl_hbm, idx_hbm, out_hbm, idx_v, bufa, bufb, zbuf, sema, semb):
    wid = lax.axis_index("s") * _NC + lax.axis_index("c")
    base = wid * _B_PER_W
    pltpu.sync_copy(idx_hbm.at[pl.ds(base, _B_PER_W)], idx_v)
    lane = lax.iota(jnp.int32, 16)

    def fire(g, buf, sem):
        iv = idx_v[pl.ds(g * _CHUNK, _CHUNK)]
        t = lax.shift_right_logical(iv, 3)
        cps = []
        for l in range(_CHUNK):
            tl = jnp.sum(jnp.where(lane == l, t, 0))
            cps.append(pltpu.async_copy(tbl_hbm.at[tl], buf.at[l], sem))
        return iv, cps

    def extract(g, iv, buf):
        r = iv & 7
        for l in range(_CHUNK):
            rl = jnp.sum(jnp.where(lane == l, r, 0))
            row = g * _CHUNK + l
            for k in range(EMBED_DIM // 16):
                zbuf[row, pl.ds(k * 16, 16)] = buf[l, rl, pl.ds(k * 16, 16)]

    def step(i, carry):
        ga = 2 * i
        gb = 2 * i + 1
        iva, cpa = fire(ga, bufa, sema)
        ivb, cpb = fire(gb, bufb, semb)
        for cp in cpa:
            cp.wait()
        extract(ga, iva, bufa)
        for cp in cpb:
            cp.wait()
        extract(gb, ivb, bufb)
        return carry

    lax.fori_loop(0, _NPAIR, step, 0)
    pltpu.sync_copy(zbuf, out_hbm.at[pl.ds(base, _B_PER_W)])


_sc_gather = pl.kernel(
    _gather_body,
    out_type=jax.ShapeDtypeStruct((BATCH, EMBED_DIM), jnp.float32),
    mesh=plsc.VectorSubcoreMesh(
        core_axis_name="c", subcore_axis_name="s", num_cores=_NC, num_subcores=_NS
    ),
    scratch_types=[
        pltpu.VMEM((_B_PER_W,), jnp.int32),
        pltpu.VMEM((_CHUNK, _TILE, EMBED_DIM), jnp.float32),
        pltpu.VMEM((_CHUNK, _TILE, EMBED_DIM), jnp.float32),
        pltpu.VMEM((_B_PER_W, EMBED_DIM), jnp.float32),
        pltpu.SemaphoreType.DMA,
        pltpu.SemaphoreType.DMA,
    ],
    compiler_params=pltpu.CompilerParams(needs_layout_passes=False),
)

_MLP_BLOCK = 2048
_GRID = BATCH // _MLP_BLOCK


def _mlp_body(z_ref, w1_ref, b1_ref, w2_ref, b2_ref, o_ref):
    h = jnp.dot(z_ref[...], w1_ref[...], preferred_element_type=jnp.float32)
    h = jnp.maximum(h + b1_ref[...], 0.0)
    o_ref[...] = (
        jnp.dot(h, w2_ref[...], preferred_element_type=jnp.float32) + b2_ref[...]
    )


def _tc_mlp(z, W1, b1, W2, b2):
    return pl.pallas_call(
        _mlp_body,
        grid=(_GRID,),
        in_specs=[
            pl.BlockSpec((_MLP_BLOCK, EMBED_DIM), lambda i: (i, 0)),
            pl.BlockSpec((EMBED_DIM, HIDDEN_DIM), lambda i: (0, 0)),
            pl.BlockSpec((1, HIDDEN_DIM), lambda i: (0, 0)),
            pl.BlockSpec((HIDDEN_DIM, OUT_DIM), lambda i: (0, 0)),
            pl.BlockSpec((1, OUT_DIM), lambda i: (0, 0)),
        ],
        out_specs=pl.BlockSpec((_MLP_BLOCK, OUT_DIM), lambda i: (i, 0)),
        out_shape=jax.ShapeDtypeStruct((BATCH, OUT_DIM), jnp.float32),
    )(z, W1, b1.reshape(1, HIDDEN_DIM), W2, b2.reshape(1, OUT_DIM))


def kernel(x, emb, W1, b1, W2, b2):
    idx = x.astype(jnp.int32)
    emb3 = emb.reshape(_NTILES, _TILE, EMBED_DIM)
    z = _sc_gather(emb3, idx)
    return _tc_mlp(z, W1, b1, W2, b2)


# speedup vs baseline: 2.4358x; 1.0534x over previous
import jax
import jax.numpy as jnp
from jax import lax
from jax.experimental import pallas as pl
from jax.experimental.pallas import tpu as pltpu
from jax.experimental.pallas import tpu_sc as plsc

VOCAB = 1000000
EMBED_DIM = 64
HIDDEN_DIM = 128
OUT_DIM = 128
BATCH = 16384

_TILE = 8
_NTILES = VOCAB // _TILE

_NC = 2
_NS = 16
_NW = _NC * _NS
_B_PER_W = BATCH // _NW
_CHUNK = 16
_NPAIR = _B_PER_W // (2 * _CHUNK)


_DEPTH = 4
_NWAVES = _B_PER_W // _CHUNK
_ZHALF = _B_PER_W // 2


def _gather_body(
    tbl_hbm, idx_hbm, out_hbm, idx_v, buf0, buf1, buf2, buf3, zbuf,
    sem0, sem1, sem2, sem3,
):
    wid = lax.axis_index("s") * _NC + lax.axis_index("c")
    base = wid * _B_PER_W
    pltpu.sync_copy(idx_hbm.at[pl.ds(base, _B_PER_W)], idx_v)
    lane = lax.iota(jnp.int32, 16)
    bufs = (buf0, buf1, buf2, buf3)
    sems = (sem0, sem1, sem2, sem3)

    def fire(w, j):
        iv = idx_v[pl.ds(w * _CHUNK, _CHUNK)]
        t = lax.shift_right_logical(iv, 3)
        for l in range(_CHUNK):
            tl = jnp.sum(jnp.where(lane == l, t, 0))
            pltpu.async_copy(tbl_hbm.at[tl], bufs[j].at[l], sems[j])

    def drain(j):
        for l in range(_CHUNK):
            pltpu.make_async_copy(tbl_hbm.at[0], bufs[j].at[l], sems[j]).wait()

    def extract(w, j):
        iv = idx_v[pl.ds(w * _CHUNK, _CHUNK)]
        r = iv & 7
        for l in range(_CHUNK):
            rl = jnp.sum(jnp.where(lane == l, r, 0))
            row = (w & (_ZHALF // _CHUNK - 1)) * _CHUNK + l
            for k in range(EMBED_DIM // 16):
                zbuf[row, pl.ds(k * 16, 16)] = bufs[j][l, rl, pl.ds(k * 16, 16)]

    for j in range(_DEPTH):
        fire(j, j)

    def step(i, carry):
        for j in range(_DEPTH):
            w = _DEPTH * i + j
            drain(j)
            extract(w, j)
            nxt = w + _DEPTH

            @pl.when(nxt < _NWAVES)
            def _():
                fire(nxt, j)

        @pl.when(i == (_NWAVES // _DEPTH) // 2 - 1)
        def _():
            pltpu.sync_copy(zbuf, out_hbm.at[pl.ds(base, _ZHALF)])

        @pl.when(i == _NWAVES // _DEPTH - 1)
        def _():
            pltpu.sync_copy(zbuf, out_hbm.at[pl.ds(base + _ZHALF, _ZHALF)])

        return carry

    lax.fori_loop(0, _NWAVES // _DEPTH, step, 0)


_sc_gather = pl.kernel(
    _gather_body,
    out_type=jax.ShapeDtypeStruct((BATCH, EMBED_DIM), jnp.float32),
    mesh=plsc.VectorSubcoreMesh(
        core_axis_name="c", subcore_axis_name="s", num_cores=_NC, num_subcores=_NS
    ),
    scratch_types=[
        pltpu.VMEM((_B_PER_W,), jnp.int32),
        pltpu.VMEM((_CHUNK, _TILE, EMBED_DIM), jnp.float32),
        pltpu.VMEM((_CHUNK, _TILE, EMBED_DIM), jnp.float32),
        pltpu.VMEM((_CHUNK, _TILE, EMBED_DIM), jnp.float32),
        pltpu.VMEM((_CHUNK, _TILE, EMBED_DIM), jnp.float32),
        pltpu.VMEM((_ZHALF, EMBED_DIM), jnp.float32),
        pltpu.SemaphoreType.DMA,
        pltpu.SemaphoreType.DMA,
        pltpu.SemaphoreType.DMA,
        pltpu.SemaphoreType.DMA,
    ],
    compiler_params=pltpu.CompilerParams(needs_layout_passes=False),
)

_MLP_BLOCK = 2048
_GRID = BATCH // _MLP_BLOCK


def _mlp_body(z_ref, w1_ref, b1_ref, w2_ref, b2_ref, o_ref):
    h = jnp.dot(z_ref[...], w1_ref[...], preferred_element_type=jnp.float32)
    h = jnp.maximum(h + b1_ref[...], 0.0)
    o_ref[...] = (
        jnp.dot(h, w2_ref[...], preferred_element_type=jnp.float32) + b2_ref[...]
    )


def _tc_mlp(z, W1, b1, W2, b2):
    return pl.pallas_call(
        _mlp_body,
        grid=(_GRID,),
        in_specs=[
            pl.BlockSpec((_MLP_BLOCK, EMBED_DIM), lambda i: (i, 0)),
            pl.BlockSpec((EMBED_DIM, HIDDEN_DIM), lambda i: (0, 0)),
            pl.BlockSpec((1, HIDDEN_DIM), lambda i: (0, 0)),
            pl.BlockSpec((HIDDEN_DIM, OUT_DIM), lambda i: (0, 0)),
            pl.BlockSpec((1, OUT_DIM), lambda i: (0, 0)),
        ],
        out_specs=pl.BlockSpec((_MLP_BLOCK, OUT_DIM), lambda i: (i, 0)),
        out_shape=jax.ShapeDtypeStruct((BATCH, OUT_DIM), jnp.float32),
    )(z, W1, b1.reshape(1, HIDDEN_DIM), W2, b2.reshape(1, OUT_DIM))


def kernel(x, emb, W1, b1, W2, b2):
    idx = x.astype(jnp.int32)
    emb3 = emb.reshape(_NTILES, _TILE, EMBED_DIM)
    z = _sc_gather(emb3, idx)
    return _tc_mlp(z, W1, b1, W2, b2)
